# Initial kernel scaffold; baseline (speedup 1.0000x reference)
import jax
import jax.numpy as jnp
from jax.experimental import pallas as pl


def kernel(x, edge_index, edge_attr, sysf, W_i, W_h, W_o, pad_token, sysf_W, sysf_b):
    raise NotImplementedError("write your pallas kernel here")


# speedup vs baseline: 2.8164x; 2.8164x over previous
import functools

import jax
import jax.numpy as jnp
from jax import lax
from jax.experimental import pallas as pl
from jax.experimental.pallas import tpu as pltpu
from jax.experimental.pallas import tpu_sc as plsc

NC = 2
NS = 16
LANES = 16
DEPTH = 3
G = 80


def _relu(v):
    return jnp.maximum(v, 0.0)


def kernel(x, edge_index, edge_attr, sysf, W_i, W_h, W_o, pad_token, sysf_W,
           sysf_b):
    N, D = x.shape
    E = edge_index.shape[1]
    B = sysf.shape[0]
    f32 = jnp.float32

    src = edge_index[0].astype(jnp.int32)
    dst = edge_index[1].astype(jnp.int32)
    src2 = src.reshape(E // G, G)
    dst2 = dst.reshape(E // G, G)

    CH = D // NC
    NR = N // NS
    RB = 800
    EC = E // NS
    RB0 = 400
    EW = E // (NC * NS)
    assert EC % RB == 0 and EW % RB0 == 0 and N % NS == 0
    assert RB % G == 0 and RB0 % G == 0 and CH % LANES == 0

    mesh = plsc.VectorSubcoreMesh(core_axis_name="c", subcore_axis_name="s")
    sc_params = pltpu.CompilerParams(use_tc_tiling_on_sc=False)


    @functools.partial(
        pl.kernel,
        out_type=jax.ShapeDtypeStruct((E, D), f32),
        mesh=mesh,
        compiler_params=sc_params,
        scratch_types=[
            pltpu.VMEM((RB0 // G, G), jnp.int32),
            pltpu.VMEM((RB0, D), f32),
            pltpu.SemaphoreType.DMA,
        ],
    )
    def sc_gather_rows(xw_hbm, src2_hbm, out_hbm, idx_v, buf_v, sem):
        wid = lax.axis_index("s") * NC + lax.axis_index("c")

        def body(i, carry):
            e0 = wid * EW + i * RB0
            r0 = wid * (EW // G) + i * (RB0 // G)
            pltpu.sync_copy(src2_hbm.at[pl.ds(r0, RB0 // G)], idx_v)
            for j in range(RB0 // G):
                pltpu.async_copy(xw_hbm.at[idx_v.at[j]],
                                 buf_v.at[pl.ds(j * G, G)], sem).wait()
            pltpu.sync_copy(buf_v, out_hbm.at[pl.ds(e0, RB0)])
            return carry

        lax.fori_loop(0, EW // RB0, body, 0)

    def _zero_accum(a_sh, zbuf, sid):
        def zb(i, carry):
            zbuf[i, :] = jnp.zeros((LANES,), f32)
            return carry

        lax.fori_loop(0, NR, zb, 0)
        for cg in range(CH // LANES):
            pltpu.sync_copy(
                zbuf, a_sh.at[pl.ds(sid * NR, NR), pl.ds(cg * LANES, LANES)])

    def _scatter_add(h_hbm, dst2_hbm, a_sh, idx_v, hbuf, sid, c0):
        def body(i, carry):
            e0 = sid * EC + i * RB
            r0 = sid * (EC // G) + i * (RB // G)
            pltpu.sync_copy(dst2_hbm.at[pl.ds(r0, RB // G)], idx_v)
            pltpu.sync_copy(h_hbm.at[pl.ds(e0, RB), pl.ds(c0, CH)], hbuf)
            for j in range(RB // G):
                pltpu.sync_copy(hbuf.at[pl.ds(j * G, G)],
                                a_sh.at[idx_v.at[j]], add=True)
            return carry

        lax.fori_loop(0, EC // RB, body, 0)

    depth_scratch = [
        pltpu.VMEM_SHARED((N, CH), f32),
        pltpu.VMEM((NR, LANES), f32),
        pltpu.VMEM((RB // G, G), jnp.int32),
        pltpu.VMEM((RB, CH), f32),
    ]

    @functools.partial(
        pl.kernel,
        out_type=jax.ShapeDtypeStruct((E, D), f32),
        mesh=mesh,
        compiler_params=sc_params,
        scratch_types=depth_scratch,
    )
    def sc_seg_gather(h_hbm, dst2_hbm, src2_hbm, g_hbm, a_sh, zbuf, idx_v,
                      hbuf):
        cid = lax.axis_index("c")
        sid = lax.axis_index("s")
        c0 = cid * CH
        _zero_accum(a_sh, zbuf, sid)
        plsc.subcore_barrier()
        _scatter_add(h_hbm, dst2_hbm, a_sh, idx_v, hbuf, sid, c0)
        plsc.subcore_barrier()

        def body(i, carry):
            e0 = sid * EC + i * RB
            r0 = sid * (EC // G) + i * (RB // G)
            pltpu.sync_copy(src2_hbm.at[pl.ds(r0, RB // G)], idx_v)
            for j in range(RB // G):
                pltpu.sync_copy(a_sh.at[idx_v.at[j]],
                                hbuf.at[pl.ds(j * G, G)])
            pltpu.sync_copy(hbuf, g_hbm.at[pl.ds(e0, RB), pl.ds(c0, CH)])
            return carry

        lax.fori_loop(0, EC // RB, body, 0)

    @functools.partial(
        pl.kernel,
        out_type=jax.ShapeDtypeStruct((N, D), f32),
        mesh=mesh,
        compiler_params=sc_params,
        scratch_types=depth_scratch,
    )
    def sc_seg_final(h_hbm, dst2_hbm, a_hbm, a_sh, zbuf, idx_v, hbuf):
        cid = lax.axis_index("c")
        sid = lax.axis_index("s")
        c0 = cid * CH
        _zero_accum(a_sh, zbuf, sid)
        plsc.subcore_barrier()
        _scatter_add(h_hbm, dst2_hbm, a_sh, idx_v, hbuf, sid, c0)
        plsc.subcore_barrier()
        for cg in range(CH // LANES):
            pltpu.sync_copy(
                a_sh.at[pl.ds(sid * NR, NR), pl.ds(cg * LANES, LANES)], zbuf)
            pltpu.sync_copy(
                zbuf,
                a_hbm.at[pl.ds(sid * NR, NR), pl.ds(c0 + cg * LANES, LANES)])


    NBX = 5
    BN = N // NBX
    BR = 1600
    NB = E // BR
    HB = (E // 2) // BR
    assert N % NBX == 0 and E % BR == 0 and (E // 2) % BR == 0

    def t_matmul(x_ref, w_ref, o_ref):
        o_ref[...] = jnp.dot(x_ref[...], w_ref[...],
                             preferred_element_type=f32)

    xw = pl.pallas_call(
        t_matmul,
        grid=(NBX,),
        in_specs=[pl.BlockSpec((BN, D), lambda i: (i, 0)),
                  pl.BlockSpec((D, D), lambda i: (0, 0))],
        out_specs=pl.BlockSpec((BN, D), lambda i: (i, 0)),
        out_shape=jax.ShapeDtypeStruct((N, D), f32),
    )(x, W_i[:D])

    g0 = sc_gather_rows(xw, src2)

    DE = edge_attr.shape[1]

    def t_init(g0_ref, ea_ref, w_ref, o_ref):
        o_ref[...] = _relu(g0_ref[...] +
                           jnp.dot(ea_ref[...], w_ref[...],
                                   preferred_element_type=f32))

    h0 = pl.pallas_call(
        t_init,
        grid=(NB,),
        in_specs=[pl.BlockSpec((BR, D), lambda i: (i, 0)),
                  pl.BlockSpec((BR, DE), lambda i: (i, 0)),
                  pl.BlockSpec((DE, D), lambda i: (0, 0))],
        out_specs=pl.BlockSpec((BR, D), lambda i: (i, 0)),
        out_shape=jax.ShapeDtypeStruct((E, D), f32),
    )(g0, edge_attr, W_i[D:])

    def t_step(h0_ref, g_ref, hr_ref, w_ref, o_ref):
        o_ref[...] = _relu(h0_ref[...] +
                           jnp.dot(g_ref[...] - hr_ref[...], w_ref[...],
                                   preferred_element_type=f32))

    step = pl.pallas_call(
        t_step,
        grid=(NB,),
        in_specs=[pl.BlockSpec((BR, D), lambda i: (i, 0)),
                  pl.BlockSpec((BR, D), lambda i: (i, 0)),
                  pl.BlockSpec((BR, D), lambda i: ((i + HB) % NB, 0)),
                  pl.BlockSpec((D, D), lambda i: (0, 0))],
        out_specs=pl.BlockSpec((BR, D), lambda i: (i, 0)),
        out_shape=jax.ShapeDtypeStruct((E, D), f32),
    )

    h = h0
    for _ in range(DEPTH - 1):
        g = sc_seg_gather(h, dst2, src2)
        h = step(h0, g, h, W_h)

    a_final = sc_seg_final(h, dst2)

    def t_out(x_ref, a_ref, wx_ref, wa_ref, o_ref):
        o_ref[...] = _relu(jnp.dot(x_ref[...], wx_ref[...],
                                   preferred_element_type=f32) +
                           jnp.dot(a_ref[...], wa_ref[...],
                                   preferred_element_type=f32))

    atom_h = pl.pallas_call(
        t_out,
        grid=(NBX,),
        in_specs=[pl.BlockSpec((BN, D), lambda i: (i, 0)),
                  pl.BlockSpec((BN, D), lambda i: (i, 0)),
                  pl.BlockSpec((D, D), lambda i: (0, 0)),
                  pl.BlockSpec((D, D), lambda i: (0, 0))],
        out_specs=pl.BlockSpec((BN, D), lambda i: (i, 0)),
        out_shape=jax.ShapeDtypeStruct((N, D), f32),
    )(x, a_final, W_o[:D], W_o[D:])

    NSF = sysf.shape[1]

    def t_sysf(s_ref, w_ref, b_ref, o_ref):
        o_ref[...] = jnp.dot(s_ref[...], w_ref[...],
                             preferred_element_type=f32) + b_ref[...]

    sysf_out = pl.pallas_call(
        t_sysf,
        in_specs=[pl.BlockSpec((B, NSF), lambda: (0, 0)),
                  pl.BlockSpec((NSF, D), lambda: (0, 0)),
                  pl.BlockSpec((1, D), lambda: (0, 0))],
        out_specs=pl.BlockSpec((B, D), lambda: (0, 0)),
        out_shape=jax.ShapeDtypeStruct((B, D), f32),
    )(sysf, sysf_W, sysf_b.reshape(1, D))

    return (sysf_out[:, None, :], atom_h.reshape(B, N // B, D))


# speedup vs baseline: 3.6392x; 1.2922x over previous
import functools

import jax
import jax.numpy as jnp
from jax import lax
from jax.experimental import pallas as pl
from jax.experimental.pallas import tpu as pltpu
from jax.experimental.pallas import tpu_sc as plsc

NC = 2
NS = 16
LANES = 16
DEPTH = 3
G = 80
RB = 400
GPC = RB // G


def _relu(v):
    return jnp.maximum(v, 0.0)


def kernel(x, edge_index, edge_attr, sysf, W_i, W_h, W_o, pad_token, sysf_W,
           sysf_b):
    N, D = x.shape
    E = edge_index.shape[1]
    B = sysf.shape[0]
    f32 = jnp.float32

    src = edge_index[0].astype(jnp.int32)
    dst = edge_index[1].astype(jnp.int32)
    src2 = src.reshape(E // G, G)
    dst2 = dst.reshape(E // G, G)
    zrows = jnp.zeros((RB, D // NC), f32)

    CH = D // NC
    NR = N // NS
    EC = E // NS
    EW = E // (NC * NS)
    NCH = EC // RB
    NCW = EW // RB
    ZR0 = min(RB, NR)
    ZR1 = NR - ZR0
    assert EC % RB == 0 and EW % RB == 0 and N % NS == 0 and NR <= 2 * RB
    assert RB % G == 0 and CH % LANES == 0

    mesh = plsc.VectorSubcoreMesh(core_axis_name="c", subcore_axis_name="s")
    sc_params = pltpu.CompilerParams(use_tc_tiling_on_sc=False)


    @functools.partial(
        pl.kernel,
        out_type=jax.ShapeDtypeStruct((E, D), f32),
        mesh=mesh,
        compiler_params=sc_params,
        scratch_types=[
            pltpu.VMEM((EW // G, G), jnp.int32),
            pltpu.VMEM((RB, D), f32),
            pltpu.VMEM((RB, D), f32),
            pltpu.SemaphoreType.DMA,
            pltpu.SemaphoreType.DMA,
            pltpu.SemaphoreType.DMA,
            pltpu.SemaphoreType.DMA,
        ],
    )
    def sc_gather_rows(xw_hbm, src2_hbm, out_hbm, idx_v, b0, b1, sg0, sg1,
                       sw0, sw1):
        wid = lax.axis_index("s") * NC + lax.axis_index("c")
        bufs, sgs, sws = (b0, b1), (sg0, sg1), (sw0, sw1)
        pltpu.sync_copy(src2_hbm.at[pl.ds(wid * (EW // G), EW // G)], idx_v)

        def body(o, carry):
            for b in (0, 1):
                i = o * 2 + b

                @pl.when(i < NCW)
                def _():
                    e0 = wid * EW + i * RB

                    @pl.when(o >= 1)
                    def _():
                        pltpu.make_async_copy(
                            bufs[b], out_hbm.at[pl.ds(e0, RB)],
                            sws[b]).wait()

                    for j in range(GPC):
                        pltpu.async_copy(xw_hbm.at[idx_v.at[i * GPC + j]],
                                         bufs[b].at[pl.ds(j * G, G)], sgs[b])
                    pltpu.make_async_copy(xw_hbm.at[pl.ds(0, RB)], bufs[b],
                                          sgs[b]).wait()
                    pltpu.async_copy(bufs[b], out_hbm.at[pl.ds(e0, RB)],
                                     sws[b])
            return carry

        lax.fori_loop(0, (NCW + 1) // 2, body, 0)
        for b in (0, 1):
            i = NCW - 2 + b
            pltpu.make_async_copy(bufs[b],
                                  out_hbm.at[pl.ds(wid * EW + i * RB, RB)],
                                  sws[b]).wait()

    def _zero_accum(a_sh, zrows_hbm, buf, sid):
        pltpu.sync_copy(zrows_hbm, buf)
        pltpu.sync_copy(buf.at[pl.ds(0, ZR0)],
                        a_sh.at[pl.ds(sid * NR, ZR0)])
        if ZR1 > 0:
            pltpu.sync_copy(buf.at[pl.ds(0, ZR1)],
                            a_sh.at[pl.ds(sid * NR + ZR0, ZR1)])

    def _scatter_add(h_hbm, dst2_hbm, a_sh, idx_v, bufs, sls, sas, sid, c0):
        pltpu.sync_copy(dst2_hbm.at[pl.ds(sid * (EC // G), EC // G)], idx_v)
        pltpu.async_copy(h_hbm.at[pl.ds(sid * EC, RB), pl.ds(c0, CH)],
                         bufs[0], sls[0])

        def body(o, carry):
            for b in (0, 1):
                i = o * 2 + b
                e0 = sid * EC + i * RB

                @pl.when(i >= 1)
                def _():
                    pltpu.make_async_copy(
                        h_hbm.at[pl.ds(e0, RB), pl.ds(c0, CH)],
                        bufs[1 - b], sas[1 - b]).wait()

                @pl.when(i + 1 < NCH)
                def _():
                    pltpu.async_copy(
                        h_hbm.at[pl.ds(e0 + RB, RB), pl.ds(c0, CH)],
                        bufs[1 - b], sls[1 - b])

                pltpu.make_async_copy(
                    h_hbm.at[pl.ds(e0, RB), pl.ds(c0, CH)], bufs[b],
                    sls[b]).wait()
                for j in range(GPC):
                    pltpu.async_copy(bufs[b].at[pl.ds(j * G, G)],
                                     a_sh.at[idx_v.at[i * GPC + j]], sas[b],
                                     add=True)
            return carry

        lax.fori_loop(0, NCH // 2, body, 0)
        pltpu.make_async_copy(
            h_hbm.at[pl.ds(sid * EC, RB), pl.ds(c0, CH)],
            bufs[(NCH - 1) % 2], sas[(NCH - 1) % 2]).wait()

    depth_scratch = [
        pltpu.VMEM_SHARED((N, CH), f32),
        pltpu.VMEM((EC // G, G), jnp.int32),
        pltpu.VMEM((RB, CH), f32),
        pltpu.VMEM((RB, CH), f32),
        pltpu.SemaphoreType.DMA,
        pltpu.SemaphoreType.DMA,
        pltpu.SemaphoreType.DMA,
        pltpu.SemaphoreType.DMA,
    ]

    @functools.partial(
        pl.kernel,
        out_type=jax.ShapeDtypeStruct((E, D), f32),
        mesh=mesh,
        compiler_params=sc_params,
        scratch_types=depth_scratch,
    )
    def sc_seg_gather(h_hbm, dst2_hbm, src2_hbm, zrows_hbm, g_hbm, a_sh,
                      idx_v, b0, b1, s0, s1, s2, s3):
        cid = lax.axis_index("c")
        sid = lax.axis_index("s")
        c0 = cid * CH
        bufs, sls, sas = (b0, b1), (s0, s1), (s2, s3)
        _zero_accum(a_sh, zrows_hbm, b0, sid)
        plsc.subcore_barrier()
        _scatter_add(h_hbm, dst2_hbm, a_sh, idx_v, bufs, sls, sas, sid, c0)
        plsc.subcore_barrier()

        pltpu.sync_copy(src2_hbm.at[pl.ds(sid * (EC // G), EC // G)], idx_v)

        def body(o, carry):
            for b in (0, 1):
                i = o * 2 + b
                e0 = sid * EC + i * RB

                @pl.when(o >= 1)
                def _():
                    pltpu.make_async_copy(
                        bufs[b], g_hbm.at[pl.ds(e0, RB), pl.ds(c0, CH)],
                        sls[b]).wait()

                for j in range(GPC):
                    pltpu.async_copy(a_sh.at[idx_v.at[i * GPC + j]],
                                     bufs[b].at[pl.ds(j * G, G)], sas[b])
                pltpu.make_async_copy(
                    h_hbm.at[pl.ds(e0, RB), pl.ds(c0, CH)], bufs[b],
                    sas[b]).wait()
                pltpu.async_copy(bufs[b],
                                 g_hbm.at[pl.ds(e0, RB), pl.ds(c0, CH)],
                                 sls[b])
            return carry

        lax.fori_loop(0, NCH // 2, body, 0)
        for b in (0, 1):
            i = NCH - 2 + b
            pltpu.make_async_copy(
                bufs[b],
                g_hbm.at[pl.ds(sid * EC + i * RB, RB), pl.ds(c0, CH)],
                sls[b]).wait()

    @functools.partial(
        pl.kernel,
        out_type=jax.ShapeDtypeStruct((N, D), f32),
        mesh=mesh,
        compiler_params=sc_params,
        scratch_types=depth_scratch,
    )
    def sc_seg_final(h_hbm, dst2_hbm, zrows_hbm, a_hbm, a_sh, idx_v, b0, b1,
                     s0, s1, s2, s3):
        cid = lax.axis_index("c")
        sid = lax.axis_index("s")
        c0 = cid * CH
        _zero_accum(a_sh, zrows_hbm, b0, sid)
        plsc.subcore_barrier()
        _scatter_add(h_hbm, dst2_hbm, a_sh, idx_v, (b0, b1), (s0, s1),
                     (s2, s3), sid, c0)
        plsc.subcore_barrier()
        pltpu.sync_copy(a_sh.at[pl.ds(sid * NR, ZR0)], b0)
        pltpu.sync_copy(b0, a_hbm.at[pl.ds(sid * NR, ZR0), pl.ds(c0, CH)])
        if ZR1 > 0:
            pltpu.sync_copy(a_sh.at[pl.ds(sid * NR + ZR0, ZR1)],
                            b1.at[pl.ds(0, ZR1)])
            pltpu.sync_copy(b1.at[pl.ds(0, ZR1)],
                            a_hbm.at[pl.ds(sid * NR + ZR0, ZR1),
                                     pl.ds(c0, CH)])


    NBX = 5
    BN = N // NBX
    BR = 1600
    NB = E // BR
    HB = (E // 2) // BR
    assert N % NBX == 0 and E % BR == 0 and (E // 2) % BR == 0

    def t_matmul(x_ref, w_ref, o_ref):
        o_ref[...] = jnp.dot(x_ref[...], w_ref[...],
                             preferred_element_type=f32)

    xw = pl.pallas_call(
        t_matmul,
        grid=(NBX,),
        in_specs=[pl.BlockSpec((BN, D), lambda i: (i, 0)),
                  pl.BlockSpec((D, D), lambda i: (0, 0))],
        out_specs=pl.BlockSpec((BN, D), lambda i: (i, 0)),
        out_shape=jax.ShapeDtypeStruct((N, D), f32),
    )(x, W_i[:D])

    g0 = sc_gather_rows(xw, src2)

    DE = edge_attr.shape[1]

    def t_init(g0_ref, ea_ref, w_ref, o_ref):
        o_ref[...] = _relu(g0_ref[...] +
                           jnp.dot(ea_ref[...], w_ref[...],
                                   preferred_element_type=f32))

    h0 = pl.pallas_call(
        t_init,
        grid=(NB,),
        in_specs=[pl.BlockSpec((BR, D), lambda i: (i, 0)),
                  pl.BlockSpec((BR, DE), lambda i: (i, 0)),
                  pl.BlockSpec((DE, D), lambda i: (0, 0))],
        out_specs=pl.BlockSpec((BR, D), lambda i: (i, 0)),
        out_shape=jax.ShapeDtypeStruct((E, D), f32),
    )(g0, edge_attr, W_i[D:])

    def t_step(h0_ref, g_ref, hr_ref, w_ref, o_ref):
        o_ref[...] = _relu(h0_ref[...] +
                           jnp.dot(g_ref[...] - hr_ref[...], w_ref[...],
                                   preferred_element_type=f32))

    step = pl.pallas_call(
        t_step,
        grid=(NB,),
        in_specs=[pl.BlockSpec((BR, D), lambda i: (i, 0)),
                  pl.BlockSpec((BR, D), lambda i: (i, 0)),
                  pl.BlockSpec((BR, D), lambda i: ((i + HB) % NB, 0)),
                  pl.BlockSpec((D, D), lambda i: (0, 0))],
        out_specs=pl.BlockSpec((BR, D), lambda i: (i, 0)),
        out_shape=jax.ShapeDtypeStruct((E, D), f32),
    )

    h = h0
    for _ in range(DEPTH - 1):
        g = sc_seg_gather(h, dst2, src2, zrows)
        h = step(h0, g, h, W_h)

    a_final = sc_seg_final(h, dst2, zrows)

    def t_out(x_ref, a_ref, wx_ref, wa_ref, o_ref):
        o_ref[...] = _relu(jnp.dot(x_ref[...], wx_ref[...],
                                   preferred_element_type=f32) +
                           jnp.dot(a_ref[...], wa_ref[...],
                                   preferred_element_type=f32))

    atom_h = pl.pallas_call(
        t_out,
        grid=(NBX,),
        in_specs=[pl.BlockSpec((BN, D), lambda i: (i, 0)),
                  pl.BlockSpec((BN, D), lambda i: (i, 0)),
                  pl.BlockSpec((D, D), lambda i: (0, 0)),
                  pl.BlockSpec((D, D), lambda i: (0, 0))],
        out_specs=pl.BlockSpec((BN, D), lambda i: (i, 0)),
        out_shape=jax.ShapeDtypeStruct((N, D), f32),
    )(x, a_final, W_o[:D], W_o[D:])

    NSF = sysf.shape[1]

    def t_sysf(s_ref, w_ref, b_ref, o_ref):
        o_ref[...] = jnp.dot(s_ref[...], w_ref[...],
                             preferred_element_type=f32) + b_ref[...]

    sysf_out = pl.pallas_call(
        t_sysf,
        in_specs=[pl.BlockSpec((B, NSF), lambda: (0, 0)),
                  pl.BlockSpec((NSF, D), lambda: (0, 0)),
                  pl.BlockSpec((1, D), lambda: (0, 0))],
        out_specs=pl.BlockSpec((B, D), lambda: (0, 0)),
        out_shape=jax.ShapeDtypeStruct((B, D), f32),
    )(sysf, sysf_W, sysf_b.reshape(1, D))

    return (sysf_out[:, None, :], atom_h.reshape(B, N // B, D))


# speedup vs baseline: 3.8098x; 1.0469x over previous
import functools

import jax
import jax.numpy as jnp
from jax import lax
from jax.experimental import pallas as pl
from jax.experimental.pallas import tpu as pltpu
from jax.experimental.pallas import tpu_sc as plsc

NC = 2
NS = 16
LANES = 16
DEPTH = 3
G = 80
RB = 400
GPC = RB // G


def _relu(v):
    return jnp.maximum(v, 0.0)


def kernel(x, edge_index, edge_attr, sysf, W_i, W_h, W_o, pad_token, sysf_W,
           sysf_b):
    N, D = x.shape
    E = edge_index.shape[1]
    B = sysf.shape[0]
    f32 = jnp.float32

    src = edge_index[0].astype(jnp.int32)
    dst = edge_index[1].astype(jnp.int32)
    src2 = src.reshape(E // G, G)
    dst2 = dst.reshape(E // G, G)
    zrows = jnp.zeros((RB, D // NC), f32)

    CH = D // NC
    NR = N // NS
    EC = E // NS
    EW = E // (NC * NS)
    NCH = EC // RB
    NCW = EW // RB
    ZR0 = min(RB, NR)
    ZR1 = NR - ZR0
    assert EC % RB == 0 and EW % RB == 0 and N % NS == 0 and NR <= 2 * RB
    assert RB % G == 0 and CH % LANES == 0

    mesh = plsc.VectorSubcoreMesh(core_axis_name="c", subcore_axis_name="s")
    sc_params = pltpu.CompilerParams(use_tc_tiling_on_sc=False)


    def _gather_out(g_hbm, src2_hbm, a_sh, idx_v, bufs, wsems, gsems, sid,
                    c0, dummy_hbm):
        pltpu.sync_copy(src2_hbm.at[pl.ds(sid * (EC // G), EC // G)], idx_v)

        def body(o, carry):
            for b in (0, 1):
                i = o * 2 + b
                e0 = sid * EC + i * RB

                @pl.when(o >= 1)
                def _():
                    pltpu.make_async_copy(
                        bufs[b], g_hbm.at[pl.ds(e0, RB), pl.ds(c0, CH)],
                        wsems[b]).wait()

                for j in range(GPC):
                    pltpu.async_copy(a_sh.at[idx_v.at[i * GPC + j]],
                                     bufs[b].at[pl.ds(j * G, G)], gsems[b])
                pltpu.make_async_copy(
                    dummy_hbm.at[pl.ds(0, RB), pl.ds(0, CH)], bufs[b],
                    gsems[b]).wait()
                pltpu.async_copy(bufs[b],
                                 g_hbm.at[pl.ds(e0, RB), pl.ds(c0, CH)],
                                 wsems[b])
            return carry

        lax.fori_loop(0, NCH // 2, body, 0)
        for b in (0, 1):
            i = NCH - 2 + b
            pltpu.make_async_copy(
                bufs[b],
                g_hbm.at[pl.ds(sid * EC + i * RB, RB), pl.ds(c0, CH)],
                wsems[b]).wait()

    def _zero_accum(a_sh, zrows_hbm, buf, sid):
        pltpu.sync_copy(zrows_hbm, buf)
        pltpu.sync_copy(buf.at[pl.ds(0, ZR0)],
                        a_sh.at[pl.ds(sid * NR, ZR0)])
        if ZR1 > 0:
            pltpu.sync_copy(buf.at[pl.ds(0, ZR1)],
                            a_sh.at[pl.ds(sid * NR + ZR0, ZR1)])

    def _scatter_add(h_hbm, dst2_hbm, a_sh, idx_v, bufs, sls, sas, sid, c0):
        pltpu.sync_copy(dst2_hbm.at[pl.ds(sid * (EC // G), EC // G)], idx_v)
        pltpu.async_copy(h_hbm.at[pl.ds(sid * EC, RB), pl.ds(c0, CH)],
                         bufs[0], sls[0])

        def body(o, carry):
            for b in (0, 1):
                i = o * 2 + b
                e0 = sid * EC + i * RB

                @pl.when(i >= 1)
                def _():
                    pltpu.make_async_copy(
                        h_hbm.at[pl.ds(e0, RB), pl.ds(c0, CH)],
                        bufs[1 - b], sas[1 - b]).wait()

                @pl.when(i + 1 < NCH)
                def _():
                    pltpu.async_copy(
                        h_hbm.at[pl.ds(e0 + RB, RB), pl.ds(c0, CH)],
                        bufs[1 - b], sls[1 - b])

                pltpu.make_async_copy(
                    h_hbm.at[pl.ds(e0, RB), pl.ds(c0, CH)], bufs[b],
                    sls[b]).wait()
                for j in range(GPC):
                    pltpu.async_copy(bufs[b].at[pl.ds(j * G, G)],
                                     a_sh.at[idx_v.at[i * GPC + j]], sas[b],
                                     add=True)
            return carry

        lax.fori_loop(0, NCH // 2, body, 0)
        pltpu.make_async_copy(
            h_hbm.at[pl.ds(sid * EC, RB), pl.ds(c0, CH)],
            bufs[(NCH - 1) % 2], sas[(NCH - 1) % 2]).wait()

    depth_scratch = [
        pltpu.VMEM_SHARED((N, CH), f32),
        pltpu.VMEM((EC // G, G), jnp.int32),
        pltpu.VMEM((RB, CH), f32),
        pltpu.VMEM((RB, CH), f32),
        pltpu.SemaphoreType.DMA,
        pltpu.SemaphoreType.DMA,
        pltpu.SemaphoreType.DMA,
        pltpu.SemaphoreType.DMA,
    ]

    @functools.partial(
        pl.kernel,
        out_type=jax.ShapeDtypeStruct((E, D), f32),
        mesh=mesh,
        compiler_params=sc_params,
        scratch_types=depth_scratch,
    )
    def sc_gather0(xw_hbm, src2_hbm, g_hbm, a_sh, idx_v, b0, b1, s0, s1, s2,
                   s3):
        cid = lax.axis_index("c")
        sid = lax.axis_index("s")
        c0 = cid * CH
        pltpu.sync_copy(xw_hbm.at[pl.ds(sid * NR, ZR0), pl.ds(c0, CH)], b0)
        pltpu.sync_copy(b0, a_sh.at[pl.ds(sid * NR, ZR0)])
        if ZR1 > 0:
            pltpu.sync_copy(
                xw_hbm.at[pl.ds(sid * NR + ZR0, ZR1), pl.ds(c0, CH)],
                b1.at[pl.ds(0, ZR1)])
            pltpu.sync_copy(b1.at[pl.ds(0, ZR1)],
                            a_sh.at[pl.ds(sid * NR + ZR0, ZR1)])
        plsc.subcore_barrier()
        _gather_out(g_hbm, src2_hbm, a_sh, idx_v, (b0, b1), (s0, s1),
                    (s2, s3), sid, c0, xw_hbm)

    @functools.partial(
        pl.kernel,
        out_type=jax.ShapeDtypeStruct((E, D), f32),
        mesh=mesh,
        compiler_params=sc_params,
        scratch_types=depth_scratch,
    )
    def sc_seg_gather(h_hbm, dst2_hbm, src2_hbm, zrows_hbm, g_hbm, a_sh,
                      idx_v, b0, b1, s0, s1, s2, s3):
        cid = lax.axis_index("c")
        sid = lax.axis_index("s")
        c0 = cid * CH
        bufs, sls, sas = (b0, b1), (s0, s1), (s2, s3)
        _zero_accum(a_sh, zrows_hbm, b0, sid)
        plsc.subcore_barrier()
        _scatter_add(h_hbm, dst2_hbm, a_sh, idx_v, bufs, sls, sas, sid, c0)
        plsc.subcore_barrier()
        _gather_out(g_hbm, src2_hbm, a_sh, idx_v, bufs, sls, sas, sid, c0,
                    h_hbm)

    @functools.partial(
        pl.kernel,
        out_type=jax.ShapeDtypeStruct((N, D), f32),
        mesh=mesh,
        compiler_params=sc_params,
        scratch_types=depth_scratch,
    )
    def sc_seg_final(h_hbm, dst2_hbm, zrows_hbm, a_hbm, a_sh, idx_v, b0, b1,
                     s0, s1, s2, s3):
        cid = lax.axis_index("c")
        sid = lax.axis_index("s")
        c0 = cid * CH
        _zero_accum(a_sh, zrows_hbm, b0, sid)
        plsc.subcore_barrier()
        _scatter_add(h_hbm, dst2_hbm, a_sh, idx_v, (b0, b1), (s0, s1),
                     (s2, s3), sid, c0)
        plsc.subcore_barrier()
        pltpu.sync_copy(a_sh.at[pl.ds(sid * NR, ZR0)], b0)
        pltpu.sync_copy(b0, a_hbm.at[pl.ds(sid * NR, ZR0), pl.ds(c0, CH)])
        if ZR1 > 0:
            pltpu.sync_copy(a_sh.at[pl.ds(sid * NR + ZR0, ZR1)],
                            b1.at[pl.ds(0, ZR1)])
            pltpu.sync_copy(b1.at[pl.ds(0, ZR1)],
                            a_hbm.at[pl.ds(sid * NR + ZR0, ZR1),
                                     pl.ds(c0, CH)])


    NBX = 5
    BN = N // NBX
    BR = 1600
    NB = E // BR
    HB = (E // 2) // BR
    assert N % NBX == 0 and E % BR == 0 and (E // 2) % BR == 0

    def t_matmul(x_ref, w_ref, o_ref):
        o_ref[...] = jnp.dot(x_ref[...], w_ref[...],
                             preferred_element_type=f32)

    xw = pl.pallas_call(
        t_matmul,
        grid=(NBX,),
        in_specs=[pl.BlockSpec((BN, D), lambda i: (i, 0)),
                  pl.BlockSpec((D, D), lambda i: (0, 0))],
        out_specs=pl.BlockSpec((BN, D), lambda i: (i, 0)),
        out_shape=jax.ShapeDtypeStruct((N, D), f32),
    )(x, W_i[:D])

    g0 = sc_gather0(xw, src2)

    DE = edge_attr.shape[1]

    def t_init(g0_ref, ea_ref, w_ref, o_ref):
        o_ref[...] = _relu(g0_ref[...] +
                           jnp.dot(ea_ref[...], w_ref[...],
                                   preferred_element_type=f32))

    h0 = pl.pallas_call(
        t_init,
        grid=(NB,),
        in_specs=[pl.BlockSpec((BR, D), lambda i: (i, 0)),
                  pl.BlockSpec((BR, DE), lambda i: (i, 0)),
                  pl.BlockSpec((DE, D), lambda i: (0, 0))],
        out_specs=pl.BlockSpec((BR, D), lambda i: (i, 0)),
        out_shape=jax.ShapeDtypeStruct((E, D), f32),
    )(g0, edge_attr, W_i[D:])

    def t_step(h0_ref, g_ref, hr_ref, w_ref, o_ref):
        o_ref[...] = _relu(h0_ref[...] +
                           jnp.dot(g_ref[...] - hr_ref[...], w_ref[...],
                                   preferred_element_type=f32))

    step = pl.pallas_call(
        t_step,
        grid=(NB,),
        in_specs=[pl.BlockSpec((BR, D), lambda i: (i, 0)),
                  pl.BlockSpec((BR, D), lambda i: (i, 0)),
                  pl.BlockSpec((BR, D), lambda i: ((i + HB) % NB, 0)),
                  pl.BlockSpec((D, D), lambda i: (0, 0))],
        out_specs=pl.BlockSpec((BR, D), lambda i: (i, 0)),
        out_shape=jax.ShapeDtypeStruct((E, D), f32),
    )

    h = h0
    for _ in range(DEPTH - 1):
        g = sc_seg_gather(h, dst2, src2, zrows)
        h = step(h0, g, h, W_h)

    a_final = sc_seg_final(h, dst2, zrows)

    def t_out(x_ref, a_ref, wx_ref, wa_ref, o_ref):
        o_ref[...] = _relu(jnp.dot(x_ref[...], wx_ref[...],
                                   preferred_element_type=f32) +
                           jnp.dot(a_ref[...], wa_ref[...],
                                   preferred_element_type=f32))

    atom_h = pl.pallas_call(
        t_out,
        grid=(NBX,),
        in_specs=[pl.BlockSpec((BN, D), lambda i: (i, 0)),
                  pl.BlockSpec((BN, D), lambda i: (i, 0)),
                  pl.BlockSpec((D, D), lambda i: (0, 0)),
                  pl.BlockSpec((D, D), lambda i: (0, 0))],
        out_specs=pl.BlockSpec((BN, D), lambda i: (i, 0)),
        out_shape=jax.ShapeDtypeStruct((N, D), f32),
    )(x, a_final, W_o[:D], W_o[D:])

    NSF = sysf.shape[1]

    def t_sysf(s_ref, w_ref, b_ref, o_ref):
        o_ref[...] = jnp.dot(s_ref[...], w_ref[...],
                             preferred_element_type=f32) + b_ref[...]

    sysf_out = pl.pallas_call(
        t_sysf,
        in_specs=[pl.BlockSpec((B, NSF), lambda: (0, 0)),
                  pl.BlockSpec((NSF, D), lambda: (0, 0)),
                  pl.BlockSpec((1, D), lambda: (0, 0))],
        out_specs=pl.BlockSpec((B, D), lambda: (0, 0)),
        out_shape=jax.ShapeDtypeStruct((B, D), f32),
    )(sysf, sysf_W, sysf_b.reshape(1, D))

    return (sysf_out[:, None, :], atom_h.reshape(B, N // B, D))


# speedup vs baseline: 4.3632x; 1.1453x over previous
import functools

import jax
import jax.numpy as jnp
from jax import lax
from jax.experimental import pallas as pl
from jax.experimental.pallas import tpu as pltpu
from jax.experimental.pallas import tpu_sc as plsc

NC = 2
NS = 16
LANES = 16
DEPTH = 3
G = 80
RB = 400
GPC = RB // G


def _relu(v):
    return jnp.maximum(v, 0.0)


def kernel(x, edge_index, edge_attr, sysf, W_i, W_h, W_o, pad_token, sysf_W,
           sysf_b):
    N, D = x.shape
    E = edge_index.shape[1]
    B = sysf.shape[0]
    f32 = jnp.float32

    src = edge_index[0].astype(jnp.int32)
    dst = edge_index[1].astype(jnp.int32)
    src2 = src.reshape(E // G, G)
    dst2 = dst.reshape(E // G, G)
    zrows = jnp.zeros((RB, D // NC), f32)

    CH = D // NC
    NR = N // NS
    EC = E // NS
    EW = E // (NC * NS)
    NCH = EC // RB
    NCW = EW // RB
    ZR0 = min(RB, NR)
    ZR1 = NR - ZR0
    assert EC % RB == 0 and EW % RB == 0 and N % NS == 0 and NR <= 2 * RB
    assert RB % G == 0 and CH % LANES == 0

    mesh = plsc.VectorSubcoreMesh(core_axis_name="c", subcore_axis_name="s")
    sc_params = pltpu.CompilerParams(use_tc_tiling_on_sc=False)


    def _gather_out(g_hbm, src2_hbm, a_sh, idx_v, bufs, wsems, gsems, sid,
                    c0, dummy_hbm):
        pltpu.sync_copy(src2_hbm.at[pl.ds(sid * (EC // G), EC // G)], idx_v)

        def body(o, carry):
            for b in (0, 1):
                i = o * 2 + b
                e0 = sid * EC + i * RB

                @pl.when(o >= 1)
                def _():
                    pltpu.make_async_copy(
                        bufs[b], g_hbm.at[pl.ds(e0, RB), pl.ds(c0, CH)],
                        wsems[b]).wait()

                for j in range(GPC):
                    pltpu.async_copy(a_sh.at[idx_v.at[i * GPC + j]],
                                     bufs[b].at[pl.ds(j * G, G)], gsems[b])
                pltpu.make_async_copy(
                    dummy_hbm.at[pl.ds(0, RB), pl.ds(0, CH)], bufs[b],
                    gsems[b]).wait()
                pltpu.async_copy(bufs[b],
                                 g_hbm.at[pl.ds(e0, RB), pl.ds(c0, CH)],
                                 wsems[b])
            return carry

        lax.fori_loop(0, NCH // 2, body, 0)
        for b in (0, 1):
            i = NCH - 2 + b
            pltpu.make_async_copy(
                bufs[b],
                g_hbm.at[pl.ds(sid * EC + i * RB, RB), pl.ds(c0, CH)],
                wsems[b]).wait()

    def _zero_accum(a_sh, zrows_hbm, buf, sid):
        pltpu.sync_copy(zrows_hbm, buf)
        pltpu.sync_copy(buf.at[pl.ds(0, ZR0)],
                        a_sh.at[pl.ds(sid * NR, ZR0)])
        if ZR1 > 0:
            pltpu.sync_copy(buf.at[pl.ds(0, ZR1)],
                            a_sh.at[pl.ds(sid * NR + ZR0, ZR1)])

    def _scatter_add(h_hbm, dst2_hbm, a_sh, idx_v, bufs, sls, sas, sid, c0):
        pltpu.sync_copy(dst2_hbm.at[pl.ds(sid * (EC // G), EC // G)], idx_v)
        pltpu.async_copy(h_hbm.at[pl.ds(sid * EC, RB), pl.ds(c0, CH)],
                         bufs[0], sls[0])

        def body(o, carry):
            for b in (0, 1):
                i = o * 2 + b
                e0 = sid * EC + i * RB

                @pl.when(i >= 1)
                def _():
                    pltpu.make_async_copy(
                        h_hbm.at[pl.ds(e0, RB), pl.ds(c0, CH)],
                        bufs[1 - b], sas[1 - b]).wait()

                @pl.when(i + 1 < NCH)
                def _():
                    pltpu.async_copy(
                        h_hbm.at[pl.ds(e0 + RB, RB), pl.ds(c0, CH)],
                        bufs[1 - b], sls[1 - b])

                pltpu.make_async_copy(
                    h_hbm.at[pl.ds(e0, RB), pl.ds(c0, CH)], bufs[b],
                    sls[b]).wait()
                for j in range(GPC):
                    pltpu.async_copy(bufs[b].at[pl.ds(j * G, G)],
                                     a_sh.at[idx_v.at[i * GPC + j]], sas[b],
                                     add=True)
            return carry

        lax.fori_loop(0, NCH // 2, body, 0)
        pltpu.make_async_copy(
            h_hbm.at[pl.ds(sid * EC, RB), pl.ds(c0, CH)],
            bufs[(NCH - 1) % 2], sas[(NCH - 1) % 2]).wait()

    depth_scratch = [
        pltpu.VMEM_SHARED((N, CH), f32),
        pltpu.VMEM((EC // G, G), jnp.int32),
        pltpu.VMEM((RB, CH), f32),
        pltpu.VMEM((RB, CH), f32),
        pltpu.SemaphoreType.DMA,
        pltpu.SemaphoreType.DMA,
        pltpu.SemaphoreType.DMA,
        pltpu.SemaphoreType.DMA,
    ]

    @functools.partial(
        pl.kernel,
        out_type=jax.ShapeDtypeStruct((E, D), f32),
        mesh=mesh,
        compiler_params=sc_params,
        scratch_types=depth_scratch,
    )
    def sc_gather0(xw_hbm, src2_hbm, g_hbm, a_sh, idx_v, b0, b1, s0, s1, s2,
                   s3):
        cid = lax.axis_index("c")
        sid = lax.axis_index("s")
        c0 = cid * CH
        pltpu.sync_copy(xw_hbm.at[pl.ds(sid * NR, ZR0), pl.ds(c0, CH)], b0)
        pltpu.sync_copy(b0, a_sh.at[pl.ds(sid * NR, ZR0)])
        if ZR1 > 0:
            pltpu.sync_copy(
                xw_hbm.at[pl.ds(sid * NR + ZR0, ZR1), pl.ds(c0, CH)],
                b1.at[pl.ds(0, ZR1)])
            pltpu.sync_copy(b1.at[pl.ds(0, ZR1)],
                            a_sh.at[pl.ds(sid * NR + ZR0, ZR1)])
        plsc.subcore_barrier()
        _gather_out(g_hbm, src2_hbm, a_sh, idx_v, (b0, b1), (s0, s1),
                    (s2, s3), sid, c0, xw_hbm)

    @functools.partial(
        pl.kernel,
        out_type=jax.ShapeDtypeStruct((E, D), f32),
        mesh=mesh,
        compiler_params=sc_params,
        scratch_types=depth_scratch,
    )
    def sc_seg_gather(h_hbm, dst2_hbm, src2_hbm, zrows_hbm, g_hbm, a_sh,
                      idx_v, b0, b1, s0, s1, s2, s3):
        cid = lax.axis_index("c")
        sid = lax.axis_index("s")
        c0 = cid * CH
        bufs, sls, sas = (b0, b1), (s0, s1), (s2, s3)
        _zero_accum(a_sh, zrows_hbm, b0, sid)
        plsc.subcore_barrier()
        _scatter_add(h_hbm, dst2_hbm, a_sh, idx_v, bufs, sls, sas, sid, c0)
        plsc.subcore_barrier()
        _gather_out(g_hbm, src2_hbm, a_sh, idx_v, bufs, sls, sas, sid, c0,
                    h_hbm)

    @functools.partial(
        pl.kernel,
        out_type=jax.ShapeDtypeStruct((N, D), f32),
        mesh=mesh,
        compiler_params=sc_params,
        scratch_types=depth_scratch,
    )
    def sc_seg_final(h_hbm, dst2_hbm, zrows_hbm, a_hbm, a_sh, idx_v, b0, b1,
                     s0, s1, s2, s3):
        cid = lax.axis_index("c")
        sid = lax.axis_index("s")
        c0 = cid * CH
        _zero_accum(a_sh, zrows_hbm, b0, sid)
        plsc.subcore_barrier()
        _scatter_add(h_hbm, dst2_hbm, a_sh, idx_v, (b0, b1), (s0, s1),
                     (s2, s3), sid, c0)
        plsc.subcore_barrier()
        pltpu.sync_copy(a_sh.at[pl.ds(sid * NR, ZR0)], b0)
        pltpu.sync_copy(b0, a_hbm.at[pl.ds(sid * NR, ZR0), pl.ds(c0, CH)])
        if ZR1 > 0:
            pltpu.sync_copy(a_sh.at[pl.ds(sid * NR + ZR0, ZR1)],
                            b1.at[pl.ds(0, ZR1)])
            pltpu.sync_copy(b1.at[pl.ds(0, ZR1)],
                            a_hbm.at[pl.ds(sid * NR + ZR0, ZR1),
                                     pl.ds(c0, CH)])


    NBX = 5
    BN = N // NBX
    BR = 6400
    NB = E // BR
    HB = (E // 2) // BR
    assert N % NBX == 0 and E % BR == 0 and (E // 2) % BR == 0

    def t_matmul(x_ref, w_ref, o_ref):
        o_ref[...] = jnp.dot(x_ref[...], w_ref[...],
                             preferred_element_type=f32)

    xw = pl.pallas_call(
        t_matmul,
        grid=(NBX,),
        in_specs=[pl.BlockSpec((BN, D), lambda i: (i, 0)),
                  pl.BlockSpec((D, D), lambda i: (0, 0))],
        out_specs=pl.BlockSpec((BN, D), lambda i: (i, 0)),
        out_shape=jax.ShapeDtypeStruct((N, D), f32),
    )(x, W_i[:D])

    g0 = sc_gather0(xw, src2)

    DE = edge_attr.shape[1]

    def t_init(g0_ref, ea_ref, w_ref, o_ref):
        o_ref[...] = _relu(g0_ref[...] +
                           jnp.dot(ea_ref[...], w_ref[...],
                                   preferred_element_type=f32))

    h0 = pl.pallas_call(
        t_init,
        grid=(NB,),
        in_specs=[pl.BlockSpec((BR, D), lambda i: (i, 0)),
                  pl.BlockSpec((BR, DE), lambda i: (i, 0)),
                  pl.BlockSpec((DE, D), lambda i: (0, 0))],
        out_specs=pl.BlockSpec((BR, D), lambda i: (i, 0)),
        out_shape=jax.ShapeDtypeStruct((E, D), f32),
    )(g0, edge_attr, W_i[D:])

    def t_step(h0_ref, g_ref, hr_ref, w_ref, o_ref):
        o_ref[...] = _relu(h0_ref[...] +
                           jnp.dot(g_ref[...] - hr_ref[...], w_ref[...],
                                   preferred_element_type=f32))

    step = pl.pallas_call(
        t_step,
        grid=(NB,),
        in_specs=[pl.BlockSpec((BR, D), lambda i: (i, 0)),
                  pl.BlockSpec((BR, D), lambda i: (i, 0)),
                  pl.BlockSpec((BR, D), lambda i: ((i + HB) % NB, 0)),
                  pl.BlockSpec((D, D), lambda i: (0, 0))],
        out_specs=pl.BlockSpec((BR, D), lambda i: (i, 0)),
        out_shape=jax.ShapeDtypeStruct((E, D), f32),
    )

    h = h0
    for _ in range(DEPTH - 1):
        g = sc_seg_gather(h, dst2, src2, zrows)
        h = step(h0, g, h, W_h)

    a_final = sc_seg_final(h, dst2, zrows)

    def t_out(x_ref, a_ref, wx_ref, wa_ref, o_ref):
        o_ref[...] = _relu(jnp.dot(x_ref[...], wx_ref[...],
                                   preferred_element_type=f32) +
                           jnp.dot(a_ref[...], wa_ref[...],
                                   preferred_element_type=f32))

    atom_h = pl.pallas_call(
        t_out,
        grid=(NBX,),
        in_specs=[pl.BlockSpec((BN, D), lambda i: (i, 0)),
                  pl.BlockSpec((BN, D), lambda i: (i, 0)),
                  pl.BlockSpec((D, D), lambda i: (0, 0)),
                  pl.BlockSpec((D, D), lambda i: (0, 0))],
        out_specs=pl.BlockSpec((BN, D), lambda i: (i, 0)),
        out_shape=jax.ShapeDtypeStruct((N, D), f32),
    )(x, a_final, W_o[:D], W_o[D:])

    NSF = sysf.shape[1]

    def t_sysf(s_ref, w_ref, b_ref, o_ref):
        o_ref[...] = jnp.dot(s_ref[...], w_ref[...],
                             preferred_element_type=f32) + b_ref[...]

    sysf_out = pl.pallas_call(
        t_sysf,
        in_specs=[pl.BlockSpec((B, NSF), lambda: (0, 0)),
                  pl.BlockSpec((NSF, D), lambda: (0, 0)),
                  pl.BlockSpec((1, D), lambda: (0, 0))],
        out_specs=pl.BlockSpec((B, D), lambda: (0, 0)),
        out_shape=jax.ShapeDtypeStruct((B, D), f32),
    )(sysf, sysf_W, sysf_b.reshape(1, D))

    return (sysf_out[:, None, :], atom_h.reshape(B, N // B, D))
